# baseline (speedup 1.0000x reference)
import functools

import jax
import jax.numpy as jnp
from jax.experimental import pallas as pl
from jax.experimental.pallas import tpu as pltpu

B, N, DIM = 2, 2048, 768
E, TOP_K = 8, 2
HID = 4 * DIM
T_BINS = 16
KEY_TOP_M = 3
KEY_ALPHA = 0.5

TT = 256
HT = 512

_F32 = jnp.float32


def _gelu(v):
    return 0.5 * v * (1.0 + jax.lax.erf(v * 0.7071067811865476))


def _dot(a, b, precision=None):
    return jax.lax.dot_general(a, b, (((1,), (0,)), ((), ())),
                               preferred_element_type=_F32,
                               precision=precision)


def _routing_kernel(tid_ref, x_ref, wg_ref, wt_ref, dw_ref, eid_ref, xbf_ref):
    x = x_ref[0]
    tid = tid_ref[0, 0]
    n = x.shape[0]

    bins = jax.lax.broadcasted_iota(jnp.int32, (n, T_BINS), 1)
    oh = (tid[:, None] == bins).astype(_F32)
    token_sum = jax.lax.dot_general(oh, x, (((0,), (0,)), ((), ())),
                                    preferred_element_type=_F32)
    cnt = jnp.sum(oh, axis=0)[:, None]
    h_t = token_sum / (cnt + 1e-6)
    valid = cnt > 0
    valid_f = valid.astype(_F32)
    g_app = (jnp.sum(h_t * valid_f, axis=0, keepdims=True)
             / jnp.clip(jnp.sum(valid_f), 1.0, None))
    s_global = jnp.sum(jnp.abs(h_t - g_app), axis=1)[:, None]
    h_prev = jnp.concatenate([h_t[T_BINS - 1:], h_t[:T_BINS - 1]], axis=0)
    cnt_prev = jnp.concatenate(
        [jnp.zeros((1, 1), _F32), cnt[:T_BINS - 1]], axis=0)
    valid_prev = cnt_prev > 0
    s_diff = (jnp.sum(jnp.abs(h_t - h_prev), axis=1)[:, None]
              * (valid_f * valid_prev.astype(_F32)))
    s = KEY_ALPHA * s_global + (1.0 - KEY_ALPHA) * s_diff
    s = jnp.where(valid, s, -1e9)

    iota_t = jax.lax.broadcasted_iota(jnp.int32, (T_BINS, 1), 0)
    cur = s
    vals, idxs = [], []
    for _ in range(KEY_TOP_M):
        v = jnp.max(cur)
        idx = jnp.min(jnp.where(cur == v, iota_t, T_BINS))
        vals.append(v)
        idxs.append(idx)
        cur = jnp.where(iota_t == idx, -jnp.inf, cur)
    exps = [jnp.exp(v - vals[0]) for v in vals]
    z = exps[0] + exps[1] + exps[2]
    g_key = jnp.zeros((1, x.shape[1]), _F32)
    for m in range(KEY_TOP_M):
        row = jnp.sum(h_t * (iota_t == idxs[m]).astype(_F32), axis=0,
                      keepdims=True)
        g_key = g_key + (exps[m] / z) * row

    xv = jnp.concatenate([g_app, g_key], axis=1)
    logits = _dot(xv, wg_ref[...])
    iota_e = jax.lax.broadcasted_iota(jnp.int32, (1, E), 1)
    v0 = jnp.max(logits)
    e0 = jnp.min(jnp.where(logits == v0, iota_e, E))
    l_m = jnp.where(iota_e == e0, -jnp.inf, logits)
    v1 = jnp.max(l_m)
    e1 = jnp.min(jnp.where(l_m == v1, iota_e, E))

    tl = _dot(x, wt_ref[...])
    l0 = jnp.sum(tl * (iota_e == e0).astype(_F32), axis=1, keepdims=True)
    l1 = jnp.sum(tl * (iota_e == e1).astype(_F32), axis=1, keepdims=True)
    mx = jnp.maximum(l0, l1)
    p0 = jnp.exp(l0 - mx)
    p1 = jnp.exp(l1 - mx)
    zt = p0 + p1
    p0 = p0 / zt
    p1 = p1 / zt
    cond0 = (p0 > p1) | ((p0 == p1) & (e0 < e1))
    sel = jnp.where(cond0, p0, p1)
    chosen = jnp.where(cond0, e0, e1)
    dw = (iota_e == chosen).astype(_F32) * sel

    dw_ref[0] = dw
    xbf_ref[0] = x.astype(jnp.bfloat16)
    b = pl.program_id(0)
    eid_ref[b, 0] = e0
    eid_ref[b, 1] = e1


def _ffn_kernel(eid_ref, x_ref, dw_ref,
                w1a_ref, b1a_ref, w2a_ref, b2a_ref,
                w1b_ref, b1b_ref, w2b_ref, b2b_ref,
                wf1_ref, bf1_ref, wf2_ref, bf2_ref, out_ref):
    b = pl.program_id(0)
    h = pl.program_id(1)
    nh = pl.num_programs(1)
    bf16 = jnp.bfloat16

    @pl.when(h == 0)
    def _zero():
        out_ref[...] = jnp.zeros_like(out_ref)

    iota_e = jax.lax.broadcasted_iota(jnp.int32, (1, E), 1)
    dwt = dw_ref[0]
    wsa = jnp.sum(dwt * (iota_e == eid_ref[b, 0]).astype(_F32), axis=1,
                  keepdims=True)
    wsb = jnp.sum(dwt * (iota_e == eid_ref[b, 1]).astype(_F32), axis=1,
                  keepdims=True)

    xb = x_ref[0]
    hida = _gelu(_dot(xb, w1a_ref[0].astype(bf16)) + b1a_ref[0, 0][None, :])
    hidb = _gelu(_dot(xb, w1b_ref[0].astype(bf16)) + b1b_ref[0, 0][None, :])
    parta = _dot(hida.astype(bf16), w2a_ref[0].astype(bf16))
    partb = _dot(hidb.astype(bf16), w2b_ref[0].astype(bf16))
    out_ref[0] += wsa * parta + wsb * partb

    @pl.when(h == nh - 1)
    def _finish():
        agg = (out_ref[0] + wsa * b2a_ref[0, 0][None, :]
               + wsb * b2b_ref[0, 0][None, :])
        f = (_dot(agg.astype(bf16), wf1_ref[:DIM, :].astype(bf16))
             + _dot(dwt.astype(bf16), wf1_ref[DIM:, :].astype(bf16))
             + bf1_ref[0][None, :])
        f = _gelu(f)
        out_ref[0] = (_dot(f.astype(bf16), wf2_ref[...].astype(bf16))
                      + bf2_ref[0][None, :] + agg)


@functools.partial(jax.jit, static_argnames=())
def _impl(x, time_ids, W_gate, W_tok, W1, b1, W2, b2, Wf1, bf1, Wf2, bf2):
    tid3 = time_ids.astype(jnp.int32).reshape(B, 1, N)

    dw, eids, x_bf = pl.pallas_call(
        _routing_kernel,
        grid=(B,),
        in_specs=[
            pl.BlockSpec((1, 1, N), lambda b: (b, 0, 0)),
            pl.BlockSpec((1, N, DIM), lambda b: (b, 0, 0)),
            pl.BlockSpec((2 * DIM, E), lambda b: (0, 0)),
            pl.BlockSpec((DIM, E), lambda b: (0, 0)),
        ],
        out_specs=[
            pl.BlockSpec((1, N, E), lambda b: (b, 0, 0)),
            pl.BlockSpec((B, 2), lambda b: (0, 0),
                         memory_space=pltpu.SMEM),
            pl.BlockSpec((1, N, DIM), lambda b: (b, 0, 0)),
        ],
        out_shape=[
            jax.ShapeDtypeStruct((B, N, E), _F32),
            jax.ShapeDtypeStruct((B, 2), jnp.int32),
            jax.ShapeDtypeStruct((B, N, DIM), jnp.bfloat16),
        ],
        compiler_params=pltpu.CompilerParams(
            dimension_semantics=("arbitrary",)),
    )(tid3, x, W_gate, W_tok)

    nh = HID // HT
    b1r = b1.reshape(E, 1, HID)
    b2r = b2.reshape(E, 1, DIM)
    grid_spec = pltpu.PrefetchScalarGridSpec(
        num_scalar_prefetch=1,
        grid=(B, nh),
        in_specs=[
            pl.BlockSpec((1, N, DIM), lambda b, h, eid: (b, 0, 0)),
            pl.BlockSpec((1, N, E), lambda b, h, eid: (b, 0, 0)),
            pl.BlockSpec((1, DIM, HT), lambda b, h, eid: (eid[b, 0], 0, h)),
            pl.BlockSpec((1, 1, HT), lambda b, h, eid: (eid[b, 0], 0, h)),
            pl.BlockSpec((1, HT, DIM), lambda b, h, eid: (eid[b, 0], h, 0)),
            pl.BlockSpec((1, 1, DIM), lambda b, h, eid: (eid[b, 0], 0, 0)),
            pl.BlockSpec((1, DIM, HT), lambda b, h, eid: (eid[b, 1], 0, h)),
            pl.BlockSpec((1, 1, HT), lambda b, h, eid: (eid[b, 1], 0, h)),
            pl.BlockSpec((1, HT, DIM), lambda b, h, eid: (eid[b, 1], h, 0)),
            pl.BlockSpec((1, 1, DIM), lambda b, h, eid: (eid[b, 1], 0, 0)),
            pl.BlockSpec((DIM + E, DIM), lambda b, h, eid: (0, 0)),
            pl.BlockSpec((1, DIM), lambda b, h, eid: (0, 0)),
            pl.BlockSpec((DIM, DIM), lambda b, h, eid: (0, 0)),
            pl.BlockSpec((1, DIM), lambda b, h, eid: (0, 0)),
        ],
        out_specs=pl.BlockSpec((1, N, DIM), lambda b, h, eid: (b, 0, 0)),
    )
    out = pl.pallas_call(
        _ffn_kernel,
        grid_spec=grid_spec,
        out_shape=jax.ShapeDtypeStruct((B, N, DIM), _F32),
        compiler_params=pltpu.CompilerParams(
            dimension_semantics=("parallel", "arbitrary"),
            vmem_limit_bytes=100 * 1024 * 1024),
    )(eids, x_bf, dw, W1, b1r, W2, b2r, W1, b1r, W2, b2r,
      Wf1, bf1.reshape(1, DIM), Wf2, bf2.reshape(1, DIM))
    return out


def kernel(x, time_ids, num_time_bins, W_gate, W_tok, W1, b1, W2, b2,
           Wf1, bf1, Wf2, bf2):
    del num_time_bins
    return _impl(x, time_ids, W_gate, W_tok, W1, b1, W2, b2, Wf1, bf1,
                 Wf2, bf2)


# speedup vs baseline: 1.0184x; 1.0184x over previous
import functools

import jax
import jax.numpy as jnp
from jax.experimental import pallas as pl
from jax.experimental.pallas import tpu as pltpu

B, N, DIM = 2, 2048, 768
E, TOP_K = 8, 2
HID = 4 * DIM
T_BINS = 16
KEY_TOP_M = 3
KEY_ALPHA = 0.5

TT = 256
HT = 1024

_F32 = jnp.float32


def _gelu(v):
    return 0.5 * v * (1.0 + jax.lax.erf(v * 0.7071067811865476))


def _dot(a, b, precision=None):
    return jax.lax.dot_general(a, b, (((1,), (0,)), ((), ())),
                               preferred_element_type=_F32,
                               precision=precision)


def _routing_kernel(tid_ref, x_ref, wg_ref, wt_ref, dw_ref, eid_ref, xbf_ref):
    x = x_ref[0]
    tid = tid_ref[0, 0]
    n = x.shape[0]

    bins = jax.lax.broadcasted_iota(jnp.int32, (n, T_BINS), 1)
    oh = (tid[:, None] == bins).astype(_F32)
    token_sum = jax.lax.dot_general(oh, x, (((0,), (0,)), ((), ())),
                                    preferred_element_type=_F32)
    cnt = jnp.sum(oh, axis=0)[:, None]
    h_t = token_sum / (cnt + 1e-6)
    valid = cnt > 0
    valid_f = valid.astype(_F32)
    g_app = (jnp.sum(h_t * valid_f, axis=0, keepdims=True)
             / jnp.clip(jnp.sum(valid_f), 1.0, None))
    s_global = jnp.sum(jnp.abs(h_t - g_app), axis=1)[:, None]
    h_prev = jnp.concatenate([h_t[T_BINS - 1:], h_t[:T_BINS - 1]], axis=0)
    cnt_prev = jnp.concatenate(
        [jnp.zeros((1, 1), _F32), cnt[:T_BINS - 1]], axis=0)
    valid_prev = cnt_prev > 0
    s_diff = (jnp.sum(jnp.abs(h_t - h_prev), axis=1)[:, None]
              * (valid_f * valid_prev.astype(_F32)))
    s = KEY_ALPHA * s_global + (1.0 - KEY_ALPHA) * s_diff
    s = jnp.where(valid, s, -1e9)

    iota_t = jax.lax.broadcasted_iota(jnp.int32, (T_BINS, 1), 0)
    cur = s
    vals, idxs = [], []
    for _ in range(KEY_TOP_M):
        v = jnp.max(cur)
        idx = jnp.min(jnp.where(cur == v, iota_t, T_BINS))
        vals.append(v)
        idxs.append(idx)
        cur = jnp.where(iota_t == idx, -jnp.inf, cur)
    exps = [jnp.exp(v - vals[0]) for v in vals]
    z = exps[0] + exps[1] + exps[2]
    g_key = jnp.zeros((1, x.shape[1]), _F32)
    for m in range(KEY_TOP_M):
        row = jnp.sum(h_t * (iota_t == idxs[m]).astype(_F32), axis=0,
                      keepdims=True)
        g_key = g_key + (exps[m] / z) * row

    xv = jnp.concatenate([g_app, g_key], axis=1)
    logits = _dot(xv, wg_ref[...])
    iota_e = jax.lax.broadcasted_iota(jnp.int32, (1, E), 1)
    v0 = jnp.max(logits)
    e0 = jnp.min(jnp.where(logits == v0, iota_e, E))
    l_m = jnp.where(iota_e == e0, -jnp.inf, logits)
    v1 = jnp.max(l_m)
    e1 = jnp.min(jnp.where(l_m == v1, iota_e, E))

    tl = _dot(x, wt_ref[...])
    l0 = jnp.sum(tl * (iota_e == e0).astype(_F32), axis=1, keepdims=True)
    l1 = jnp.sum(tl * (iota_e == e1).astype(_F32), axis=1, keepdims=True)
    mx = jnp.maximum(l0, l1)
    p0 = jnp.exp(l0 - mx)
    p1 = jnp.exp(l1 - mx)
    zt = p0 + p1
    p0 = p0 / zt
    p1 = p1 / zt
    cond0 = (p0 > p1) | ((p0 == p1) & (e0 < e1))
    sel = jnp.where(cond0, p0, p1)
    chosen = jnp.where(cond0, e0, e1)
    dw = (iota_e == chosen).astype(_F32) * sel

    dw_ref[0] = dw
    xbf_ref[0] = x.astype(jnp.bfloat16)
    b = pl.program_id(0)
    eid_ref[b, 0] = e0
    eid_ref[b, 1] = e1


def _ffn_kernel(eid_ref, x_ref, dw_ref, w1_ref, b1_ref, w2_ref, b2_ref,
                wf1_ref, bf1_ref, wf2_ref, bf2_ref, out_ref):
    b = pl.program_id(0)
    s = pl.program_id(1)
    h = pl.program_id(2)
    ns = pl.num_programs(1)
    nh = pl.num_programs(2)
    bf16 = jnp.bfloat16

    @pl.when((s == 0) & (h == 0))
    def _zero():
        out_ref[...] = jnp.zeros_like(out_ref)

    xb = x_ref[0]
    hid = _dot(xb, w1_ref[0].astype(bf16)) + b1_ref[0, 0][None, :]
    hid = _gelu(hid)
    part = _dot(hid.astype(bf16), w2_ref[0].astype(bf16))

    e_s = eid_ref[b, s]
    iota_e = jax.lax.broadcasted_iota(jnp.int32, (1, E), 1)
    wslot = jnp.sum(dw_ref[0] * (iota_e == e_s).astype(_F32), axis=1,
                    keepdims=True)
    out_ref[0] += wslot * part

    @pl.when(h == nh - 1)
    def _bias():
        out_ref[0] += wslot * b2_ref[0, 0][None, :]

    @pl.when((s == ns - 1) & (h == nh - 1))
    def _fuse():
        agg = out_ref[0]
        dwt = dw_ref[0].astype(bf16)
        f = (_dot(agg.astype(bf16), wf1_ref[:DIM, :].astype(bf16))
             + _dot(dwt, wf1_ref[DIM:, :].astype(bf16))
             + bf1_ref[0][None, :])
        f = _gelu(f)
        out_ref[0] = (_dot(f.astype(bf16), wf2_ref[...].astype(bf16))
                      + bf2_ref[0][None, :] + agg)


@functools.partial(jax.jit, static_argnames=())
def _impl(x, time_ids, W_gate, W_tok, W1, b1, W2, b2, Wf1, bf1, Wf2, bf2):
    tid3 = time_ids.astype(jnp.int32).reshape(B, 1, N)

    dw, eids, x_bf = pl.pallas_call(
        _routing_kernel,
        grid=(B,),
        in_specs=[
            pl.BlockSpec((1, 1, N), lambda b: (b, 0, 0)),
            pl.BlockSpec((1, N, DIM), lambda b: (b, 0, 0)),
            pl.BlockSpec((2 * DIM, E), lambda b: (0, 0)),
            pl.BlockSpec((DIM, E), lambda b: (0, 0)),
        ],
        out_specs=[
            pl.BlockSpec((1, N, E), lambda b: (b, 0, 0)),
            pl.BlockSpec((B, 2), lambda b: (0, 0),
                         memory_space=pltpu.SMEM),
            pl.BlockSpec((1, N, DIM), lambda b: (b, 0, 0)),
        ],
        out_shape=[
            jax.ShapeDtypeStruct((B, N, E), _F32),
            jax.ShapeDtypeStruct((B, 2), jnp.int32),
            jax.ShapeDtypeStruct((B, N, DIM), jnp.bfloat16),
        ],
        compiler_params=pltpu.CompilerParams(
            dimension_semantics=("arbitrary",)),
    )(tid3, x, W_gate, W_tok)

    nh = HID // HT
    grid_spec = pltpu.PrefetchScalarGridSpec(
        num_scalar_prefetch=1,
        grid=(B, TOP_K, nh),
        in_specs=[
            pl.BlockSpec((1, N, DIM), lambda b, s, h, eid: (b, 0, 0)),
            pl.BlockSpec((1, N, E), lambda b, s, h, eid: (b, 0, 0)),
            pl.BlockSpec((1, DIM, HT), lambda b, s, h, eid: (eid[b, s], 0, h)),
            pl.BlockSpec((1, 1, HT), lambda b, s, h, eid: (eid[b, s], 0, h)),
            pl.BlockSpec((1, HT, DIM), lambda b, s, h, eid: (eid[b, s], h, 0)),
            pl.BlockSpec((1, 1, DIM), lambda b, s, h, eid: (eid[b, s], 0, 0)),
            pl.BlockSpec((DIM + E, DIM), lambda b, s, h, eid: (0, 0)),
            pl.BlockSpec((1, DIM), lambda b, s, h, eid: (0, 0)),
            pl.BlockSpec((DIM, DIM), lambda b, s, h, eid: (0, 0)),
            pl.BlockSpec((1, DIM), lambda b, s, h, eid: (0, 0)),
        ],
        out_specs=pl.BlockSpec((1, N, DIM), lambda b, s, h, eid: (b, 0, 0)),
    )
    out = pl.pallas_call(
        _ffn_kernel,
        grid_spec=grid_spec,
        out_shape=jax.ShapeDtypeStruct((B, N, DIM), _F32),
        compiler_params=pltpu.CompilerParams(
            dimension_semantics=("parallel", "arbitrary", "arbitrary")),
    )(eids, x_bf, dw, W1, b1.reshape(E, 1, HID), W2,
      b2.reshape(E, 1, DIM), Wf1, bf1.reshape(1, DIM), Wf2,
      bf2.reshape(1, DIM))
    return out


def kernel(x, time_ids, num_time_bins, W_gate, W_tok, W1, b1, W2, b2,
           Wf1, bf1, Wf2, bf2):
    del num_time_bins
    return _impl(x, time_ids, W_gate, W_tok, W1, b1, W2, b2, Wf1, bf1,
                 Wf2, bf2)


# speedup vs baseline: 1.0186x; 1.0002x over previous
import functools

import jax
import jax.numpy as jnp
from jax.experimental import pallas as pl
from jax.experimental.pallas import tpu as pltpu

B, N, DIM = 2, 2048, 768
E, TOP_K = 8, 2
HID = 4 * DIM
T_BINS = 16
KEY_TOP_M = 3
KEY_ALPHA = 0.5

HT = 1024

_F32 = jnp.float32


def _gelu(v):
    return 0.5 * v * (1.0 + jax.lax.erf(v * 0.7071067811865476))


def _dot(a, b, precision=None):
    return jax.lax.dot_general(a, b, (((1,), (0,)), ((), ())),
                               preferred_element_type=_F32,
                               precision=precision)


def _routing_kernel(tid_ref, x_ref, wg_ref, wt_ref, dw_ref, eid_ref, xbf_ref):
    x = x_ref[0]
    tid = tid_ref[0, 0]
    n = x.shape[0]

    bins = jax.lax.broadcasted_iota(jnp.int32, (n, T_BINS), 1)
    oh = (tid[:, None] == bins).astype(_F32)
    token_sum = jax.lax.dot_general(oh, x, (((0,), (0,)), ((), ())),
                                    preferred_element_type=_F32)
    cnt = jnp.sum(oh, axis=0)[:, None]
    h_t = token_sum / (cnt + 1e-6)
    valid = cnt > 0
    valid_f = valid.astype(_F32)
    g_app = (jnp.sum(h_t * valid_f, axis=0, keepdims=True)
             / jnp.clip(jnp.sum(valid_f), 1.0, None))
    s_global = jnp.sum(jnp.abs(h_t - g_app), axis=1)[:, None]
    h_prev = jnp.concatenate([h_t[T_BINS - 1:], h_t[:T_BINS - 1]], axis=0)
    cnt_prev = jnp.concatenate(
        [jnp.zeros((1, 1), _F32), cnt[:T_BINS - 1]], axis=0)
    valid_prev = cnt_prev > 0
    s_diff = (jnp.sum(jnp.abs(h_t - h_prev), axis=1)[:, None]
              * (valid_f * valid_prev.astype(_F32)))
    s = KEY_ALPHA * s_global + (1.0 - KEY_ALPHA) * s_diff
    s = jnp.where(valid, s, -1e9)

    iota_t = jax.lax.broadcasted_iota(jnp.int32, (T_BINS, 1), 0)
    cur = s
    vals, idxs = [], []
    for _ in range(KEY_TOP_M):
        v = jnp.max(cur)
        idx = jnp.min(jnp.where(cur == v, iota_t, T_BINS))
        vals.append(v)
        idxs.append(idx)
        cur = jnp.where(iota_t == idx, -jnp.inf, cur)
    exps = [jnp.exp(v - vals[0]) for v in vals]
    z = exps[0] + exps[1] + exps[2]
    g_key = jnp.zeros((1, x.shape[1]), _F32)
    for m in range(KEY_TOP_M):
        row = jnp.sum(h_t * (iota_t == idxs[m]).astype(_F32), axis=0,
                      keepdims=True)
        g_key = g_key + (exps[m] / z) * row

    xv = jnp.concatenate([g_app, g_key], axis=1)
    logits = _dot(xv, wg_ref[...])
    iota_e = jax.lax.broadcasted_iota(jnp.int32, (1, E), 1)
    v0 = jnp.max(logits)
    e0 = jnp.min(jnp.where(logits == v0, iota_e, E))
    l_m = jnp.where(iota_e == e0, -jnp.inf, logits)
    v1 = jnp.max(l_m)
    e1 = jnp.min(jnp.where(l_m == v1, iota_e, E))

    tl = _dot(x, wt_ref[...])
    l0 = jnp.sum(tl * (iota_e == e0).astype(_F32), axis=1, keepdims=True)
    l1 = jnp.sum(tl * (iota_e == e1).astype(_F32), axis=1, keepdims=True)
    mx = jnp.maximum(l0, l1)
    p0 = jnp.exp(l0 - mx)
    p1 = jnp.exp(l1 - mx)
    zt = p0 + p1
    p0 = p0 / zt
    p1 = p1 / zt
    cond0 = (p0 > p1) | ((p0 == p1) & (e0 < e1))
    sel = jnp.where(cond0, p0, p1)
    chosen = jnp.where(cond0, e0, e1)
    dw = (iota_e == chosen).astype(_F32) * sel

    dw_ref[0] = dw
    xbf_ref[0] = x.astype(jnp.bfloat16)
    b = pl.program_id(0)
    eid_ref[b, 0] = e0
    eid_ref[b, 1] = e1


def _ffn_kernel(eid_ref, x_ref, dw_ref, w1_ref, b1_ref, w2_ref, b2_ref,
                wf1_ref, bf1_ref, wf2_ref, bf2_ref, out_ref):
    b = pl.program_id(0)
    s = pl.program_id(1)
    h = pl.program_id(2)
    ns = pl.num_programs(1)
    nh = pl.num_programs(2)
    bf16 = jnp.bfloat16

    @pl.when((s == 0) & (h == 0))
    def _zero():
        out_ref[...] = jnp.zeros_like(out_ref)

    xb = x_ref[0]
    hid = _dot(xb, w1_ref[0].astype(bf16)) + b1_ref[0, 0][None, :]
    hid = _gelu(hid)
    part = _dot(hid.astype(bf16), w2_ref[0].astype(bf16))

    e_s = eid_ref[b, s]
    iota_e = jax.lax.broadcasted_iota(jnp.int32, (1, E), 1)
    wslot = jnp.sum(dw_ref[0] * (iota_e == e_s).astype(_F32), axis=1,
                    keepdims=True)
    out_ref[0] += wslot * part

    @pl.when(h == nh - 1)
    def _bias():
        out_ref[0] += wslot * b2_ref[0, 0][None, :]

    @pl.when((s == ns - 1) & (h == nh - 1))
    def _fuse():
        agg = out_ref[0]
        dwt = dw_ref[0].astype(bf16)
        f = (_dot(agg.astype(bf16), wf1_ref[:DIM, :].astype(bf16))
             + _dot(dwt, wf1_ref[DIM:, :].astype(bf16))
             + bf1_ref[0][None, :])
        f = _gelu(f)
        out_ref[0] = (_dot(f.astype(bf16), wf2_ref[...].astype(bf16))
                      + bf2_ref[0][None, :] + agg)


@functools.partial(jax.jit, static_argnames=())
def _impl(x, time_ids, W_gate, W_tok, W1, b1, W2, b2, Wf1, bf1, Wf2, bf2):
    tid3 = time_ids.astype(jnp.int32).reshape(B, 1, N)

    dw, eids, x_bf = pl.pallas_call(
        _routing_kernel,
        grid=(B,),
        in_specs=[
            pl.BlockSpec((1, 1, N), lambda b: (b, 0, 0)),
            pl.BlockSpec((1, N, DIM), lambda b: (b, 0, 0)),
            pl.BlockSpec((2 * DIM, E), lambda b: (0, 0)),
            pl.BlockSpec((DIM, E), lambda b: (0, 0)),
        ],
        out_specs=[
            pl.BlockSpec((1, N, E), lambda b: (b, 0, 0)),
            pl.BlockSpec((B, 2), lambda b: (0, 0),
                         memory_space=pltpu.SMEM),
            pl.BlockSpec((1, N, DIM), lambda b: (b, 0, 0)),
        ],
        out_shape=[
            jax.ShapeDtypeStruct((B, N, E), _F32),
            jax.ShapeDtypeStruct((B, 2), jnp.int32),
            jax.ShapeDtypeStruct((B, N, DIM), jnp.bfloat16),
        ],
        compiler_params=pltpu.CompilerParams(
            dimension_semantics=("arbitrary",)),
    )(tid3, x, W_gate, W_tok)

    nh = HID // HT
    grid_spec = pltpu.PrefetchScalarGridSpec(
        num_scalar_prefetch=1,
        grid=(B, TOP_K, nh),
        in_specs=[
            pl.BlockSpec((1, N, DIM), lambda b, s, h, eid: (b, 0, 0)),
            pl.BlockSpec((1, N, E), lambda b, s, h, eid: (b, 0, 0)),
            pl.BlockSpec((1, DIM, HT), lambda b, s, h, eid: (eid[b, s], 0, h)),
            pl.BlockSpec((1, 1, HT), lambda b, s, h, eid: (eid[b, s], 0, h)),
            pl.BlockSpec((1, HT, DIM), lambda b, s, h, eid: (eid[b, s], h, 0)),
            pl.BlockSpec((1, 1, DIM), lambda b, s, h, eid: (eid[b, s], 0, 0)),
            pl.BlockSpec((DIM + E, DIM), lambda b, s, h, eid: (0, 0)),
            pl.BlockSpec((1, DIM), lambda b, s, h, eid: (0, 0)),
            pl.BlockSpec((DIM, DIM), lambda b, s, h, eid: (0, 0)),
            pl.BlockSpec((1, DIM), lambda b, s, h, eid: (0, 0)),
        ],
        out_specs=pl.BlockSpec((1, N, DIM), lambda b, s, h, eid: (b, 0, 0)),
    )
    out = pl.pallas_call(
        _ffn_kernel,
        grid_spec=grid_spec,
        out_shape=jax.ShapeDtypeStruct((B, N, DIM), _F32),
        compiler_params=pltpu.CompilerParams(
            dimension_semantics=("parallel", "arbitrary", "arbitrary")),
    )(eids, x_bf, dw, W1, b1.reshape(E, 1, HID), W2,
      b2.reshape(E, 1, DIM), Wf1, bf1.reshape(1, DIM), Wf2,
      bf2.reshape(1, DIM))
    return out


def kernel(x, time_ids, num_time_bins, W_gate, W_tok, W1, b1, W2, b2,
           Wf1, bf1, Wf2, bf2):
    del num_time_bins
    return _impl(x, time_ids, W_gate, W_tok, W1, b1, W2, b2, Wf1, bf1,
                 Wf2, bf2)
